# baseline (speedup 1.0000x reference)
import functools

import jax
import jax.numpy as jnp
from jax import lax
from jax.experimental import pallas as pl
from jax.experimental.pallas import tpu as pltpu
from jax.experimental.pallas import tpu_sc as plsc

D = 768
B = 4
T = 4096
N = B * T

_info = plsc.get_sparse_core_info()
NC, NS, L = _info.num_cores, _info.num_subcores, _info.num_lanes
NW = NC * NS
PW_T = T // NW
CH = 32
NCHUNK = PW_T // CH
NSTEP = NCHUNK * B
NRING = 3


def _emb_body(tok_hbm, xf_hbm, pos_hbm, out_hbm, idx_v, rows, pos, gsem, ssem, psem):
    wid = lax.axis_index("s") * NC + lax.axis_index("c")
    t0 = wid * PW_T

    for b in range(B):
        pltpu.sync_copy(xf_hbm.at[pl.ds(b * T + t0, PW_T)], idx_v.at[b])

    def start_gather(s, k):
        c, b = s // B, s % B
        return pltpu.async_copy(
            tok_hbm.at[idx_v.at[b, pl.ds(c * CH, CH)]], rows[k], gsem[k])

    pcopy = [None] * 2
    pcopy[0] = pltpu.async_copy(pos_hbm.at[pl.ds(t0, CH)], pos[0], psem[0])
    gcopy = [None] * NRING
    scopy = [None] * NRING
    gcopy[0] = start_gather(0, 0)
    gcopy[1] = start_gather(1, 1)

    for s in range(NSTEP):
        k = s % NRING
        c, b = s // B, s % B
        q = c % 2
        gcopy[k].wait()
        if b == 0:
            pcopy[q].wait()
            if c + 1 < NCHUNK:
                pcopy[1 - q] = pltpu.async_copy(
                    pos_hbm.at[pl.ds(t0 + (c + 1) * CH, CH)], pos[1 - q], psem[1 - q])

        def row_body(r, carry, _k=k, _q=q):
            for j in range(D // L):
                sl = pl.ds(j * L, L)
                plsc.addupdate(rows[_k].at[r, sl], pos[_q][r, sl])
            return carry

        lax.fori_loop(0, CH, row_body, 0)

        scopy[k] = pltpu.async_copy(
            rows[k], out_hbm.at[pl.ds(b * T + t0 + c * CH, CH)], ssem[k])

        g = s + 2
        if g < NSTEP:
            kg = g % NRING
            if scopy[kg] is not None:
                scopy[kg].wait()
            gcopy[kg] = start_gather(g, kg)

    for s in (NSTEP - 2, NSTEP - 1):
        scopy[s % NRING].wait()


@functools.partial(
    pl.kernel,
    mesh=plsc.VectorSubcoreMesh(core_axis_name="c", subcore_axis_name="s"),
    out_type=jax.ShapeDtypeStruct((N, D), jnp.float32),
    scratch_types=[
        pltpu.VMEM((B, PW_T), jnp.int32),
        [pltpu.VMEM((CH, D), jnp.float32) for _ in range(NRING)],
        [pltpu.VMEM((CH, D), jnp.float32) for _ in range(2)],
        [pltpu.SemaphoreType.DMA for _ in range(NRING)],
        [pltpu.SemaphoreType.DMA for _ in range(NRING)],
        [pltpu.SemaphoreType.DMA for _ in range(2)],
    ],
)
def _emb_kernel(tok_hbm, xf_hbm, pos_hbm, out_hbm, idx_v, rows, pos, gsem, ssem, psem):
    _emb_body(tok_hbm, xf_hbm, pos_hbm, out_hbm, idx_v, rows, pos, gsem, ssem, psem)


def kernel(x, tok_table, pos_table):
    b, t = x.shape
    xf = x.reshape(-1).astype(jnp.int32)
    out = _emb_kernel(tok_table, xf, pos_table)
    return out.reshape(b, t, tok_table.shape[1])


# speedup vs baseline: 1.0168x; 1.0168x over previous
import functools

import jax
import jax.numpy as jnp
from jax import lax
from jax.experimental import pallas as pl
from jax.experimental.pallas import tpu as pltpu
from jax.experimental.pallas import tpu_sc as plsc

D = 768
B = 4
T = 4096
N = B * T

_info = plsc.get_sparse_core_info()
NC, NS, L = _info.num_cores, _info.num_subcores, _info.num_lanes
NW = NC * NS
PW_T = T // NW
CH = 32
NCHUNK = PW_T // CH
NSTEP = NCHUNK * B
NRING = 3


def _emb_body(tok_hbm, xf_hbm, pos_hbm, out_hbm, idx_v, rows, pos, gsem, ssem, psem):
    wid = lax.axis_index("s") * NC + lax.axis_index("c")
    t0 = wid * PW_T

    for b in range(B):
        pltpu.sync_copy(xf_hbm.at[pl.ds(b * T + t0, PW_T)], idx_v.at[b])

    def start_gather(s, k):
        c, b = s // B, s % B
        return pltpu.async_copy(
            tok_hbm.at[idx_v.at[b, pl.ds(c * CH, CH)]], rows[k], gsem[k])

    pcopy = [None] * 2
    pcopy[0] = pltpu.async_copy(pos_hbm.at[pl.ds(t0, CH)], pos[0], psem[0])
    gcopy = [None] * NRING
    scopy = [None] * NRING
    gcopy[0] = start_gather(0, 0)
    gcopy[1] = start_gather(1, 1)

    for s in range(NSTEP):
        k = s % NRING
        c, b = s // B, s % B
        q = c % 2
        gcopy[k].wait()
        if b == 0:
            pcopy[q].wait()
            if c + 1 < NCHUNK:
                pcopy[1 - q] = pltpu.async_copy(
                    pos_hbm.at[pl.ds(t0 + (c + 1) * CH, CH)], pos[1 - q], psem[1 - q])

        def row_body(r, carry, _k=k, _q=q):
            for j in range(D // L):
                sl = pl.ds(j * L, L)
                rows[_k][r, sl] = rows[_k][r, sl] + pos[_q][r, sl]
            return carry

        lax.fori_loop(0, CH, row_body, 0)

        scopy[k] = pltpu.async_copy(
            rows[k], out_hbm.at[pl.ds(b * T + t0 + c * CH, CH)], ssem[k])

        g = s + 2
        if g < NSTEP:
            kg = g % NRING
            if scopy[kg] is not None:
                scopy[kg].wait()
            gcopy[kg] = start_gather(g, kg)

    for s in (NSTEP - 2, NSTEP - 1):
        scopy[s % NRING].wait()


@functools.partial(
    pl.kernel,
    mesh=plsc.VectorSubcoreMesh(core_axis_name="c", subcore_axis_name="s"),
    out_type=jax.ShapeDtypeStruct((N, D), jnp.float32),
    scratch_types=[
        pltpu.VMEM((B, PW_T), jnp.int32),
        [pltpu.VMEM((CH, D), jnp.float32) for _ in range(NRING)],
        [pltpu.VMEM((CH, D), jnp.float32) for _ in range(2)],
        [pltpu.SemaphoreType.DMA for _ in range(NRING)],
        [pltpu.SemaphoreType.DMA for _ in range(NRING)],
        [pltpu.SemaphoreType.DMA for _ in range(2)],
    ],
)
def _emb_kernel(tok_hbm, xf_hbm, pos_hbm, out_hbm, idx_v, rows, pos, gsem, ssem, psem):
    _emb_body(tok_hbm, xf_hbm, pos_hbm, out_hbm, idx_v, rows, pos, gsem, ssem, psem)


def kernel(x, tok_table, pos_table):
    b, t = x.shape
    xf = x.reshape(-1).astype(jnp.int32)
    out = _emb_kernel(tok_table, xf, pos_table)
    return out.reshape(b, t, tok_table.shape[1])


# speedup vs baseline: 1.0441x; 1.0268x over previous
import functools

import jax
import jax.numpy as jnp
from jax import lax
from jax.experimental import pallas as pl
from jax.experimental.pallas import tpu as pltpu
from jax.experimental.pallas import tpu_sc as plsc

D = 768
B = 4
T = 4096

_info = plsc.get_sparse_core_info()
NC, NS, L = _info.num_cores, _info.num_subcores, _info.num_lanes
NW = NC * NS
PW_T = T // NW
CH = 32
NCHUNK = PW_T // CH
NSTEP = NCHUNK * B
NRING = 3


def _emb_body(tok_hbm, x_hbm, pos_hbm, out_hbm, idx_v, rows, pos, gsem, ssem, psem, isem):
    wid = lax.axis_index("s") * NC + lax.axis_index("c")
    t0 = wid * PW_T

    icopy = [
        pltpu.async_copy(x_hbm.at[b, pl.ds(t0, PW_T)], idx_v.at[b], isem)
        for b in range(B)
    ]

    def start_gather(s, k):
        c, b = s // B, s % B
        return pltpu.async_copy(
            tok_hbm.at[idx_v.at[b, pl.ds(c * CH, CH)]], rows[k], gsem[k])

    pcopy = [None] * 2
    pcopy[0] = pltpu.async_copy(pos_hbm.at[pl.ds(t0, CH)], pos[0], psem[0])
    for c in icopy:
        c.wait()
    gcopy = [None] * NRING
    scopy = [None] * NRING
    gcopy[0] = start_gather(0, 0)
    gcopy[1] = start_gather(1, 1)

    for s in range(NSTEP):
        k = s % NRING
        c, b = s // B, s % B
        q = c % 2
        gcopy[k].wait()
        if b == 0:
            pcopy[q].wait()
            if c + 1 < NCHUNK:
                pcopy[1 - q] = pltpu.async_copy(
                    pos_hbm.at[pl.ds(t0 + (c + 1) * CH, CH)], pos[1 - q], psem[1 - q])

        def row_body(r, carry, _k=k, _q=q):
            for j in range(D // L):
                sl = pl.ds(j * L, L)
                rows[_k][r, sl] = rows[_k][r, sl] + pos[_q][r, sl]
            return carry

        lax.fori_loop(0, CH, row_body, 0)

        scopy[k] = pltpu.async_copy(
            rows[k], out_hbm.at[b, pl.ds(t0 + c * CH, CH)], ssem[k])

        g = s + 2
        if g < NSTEP:
            kg = g % NRING
            if scopy[kg] is not None:
                scopy[kg].wait()
            gcopy[kg] = start_gather(g, kg)

    for s in (NSTEP - 2, NSTEP - 1):
        scopy[s % NRING].wait()


@functools.partial(
    pl.kernel,
    mesh=plsc.VectorSubcoreMesh(core_axis_name="c", subcore_axis_name="s"),
    out_type=jax.ShapeDtypeStruct((B, T, D), jnp.float32),
    scratch_types=[
        pltpu.VMEM((B, PW_T), jnp.int32),
        [pltpu.VMEM((CH, D), jnp.float32) for _ in range(NRING)],
        [pltpu.VMEM((CH, D), jnp.float32) for _ in range(2)],
        [pltpu.SemaphoreType.DMA for _ in range(NRING)],
        [pltpu.SemaphoreType.DMA for _ in range(NRING)],
        [pltpu.SemaphoreType.DMA for _ in range(2)],
        pltpu.SemaphoreType.DMA,
    ],
)
def _emb_kernel(tok_hbm, x_hbm, pos_hbm, out_hbm, idx_v, rows, pos, gsem, ssem, psem, isem):
    _emb_body(tok_hbm, x_hbm, pos_hbm, out_hbm, idx_v, rows, pos, gsem, ssem, psem, isem)


def kernel(x, tok_table, pos_table):
    return _emb_kernel(tok_table, x.astype(jnp.int32), pos_table)
